# baseline (speedup 1.0000x reference)
import jax
import jax.numpy as jnp
from jax import lax
from jax.experimental import pallas as pl
from jax.experimental.pallas import tpu as pltpu
from jax.experimental.pallas import tpu_sc as plsc

NUM_ENTITIES = 1000000
NUM_RELATIONS = 1000
EMBED_DIM = 64
BATCH = 16384

_info = plsc.get_sparse_core_info()
NC, NS, L = _info.num_cores, _info.num_subcores, _info.num_lanes
NW = NC * NS
RPW = BATCH // NW
G = 16
N_GROUPS = RPW // G
OUT_ROWS = 128
GROUPS_PER_OUT = OUT_ROWS // G
D_VECS = EMBED_DIM // L


def _issue(er3, ei3, rr3, ri3, eidx_v, ridx_v, g, bufset, sem):
    a_v, b_v, c_v, d_v = bufset
    e_vec = eidx_v[pl.ds(g * G, G)]
    r_vec = ridx_v[pl.ds(g * G, G)]
    for j in range(G):
        pe = e_vec[j] >> 3
        se = e_vec[j] & 7
        pr = r_vec[j] >> 3
        sr = r_vec[j] & 7
        pltpu.async_copy(er3.at[pe, se], a_v.at[j], sem)
        pltpu.async_copy(ei3.at[pe, se], b_v.at[j], sem)
        pltpu.async_copy(rr3.at[pr, sr], c_v.at[j], sem)
        pltpu.async_copy(ri3.at[pr, sr], d_v.at[j], sem)


def _drain(er3, bufset, sem):
    for buf in bufset:
        for j in range(G):
            pltpu.make_async_copy(er3.at[0, 0], buf.at[j], sem).wait()


def _compute(bufset, or_v, oi_v, g):
    a_v, b_v, c_v, d_v = bufset
    row0 = (g % GROUPS_PER_OUT) * G
    for j in range(G):
        for cb in range(D_VECS):
            sl = pl.ds(cb * L, L)
            a = a_v[j, sl]
            b = b_v[j, sl]
            cc = c_v[j, sl]
            d = d_v[j, sl]
            or_v[row0 + j, sl] = a * cc - b * d
            oi_v[row0 + j, sl] = a * d + b * cc


def _body(e1_hbm, r_hbm, er3, ei3, rr3, ri3, out_r, out_i,
          eidx_v, ridx_v,
          a0, b0, c0, d0, a1, b1, c1, d1, or_v, oi_v, sem0, sem1):
    wid = lax.axis_index("s") * NC + lax.axis_index("c")
    base = wid * RPW
    pltpu.sync_copy(e1_hbm.at[pl.ds(base, RPW)], eidx_v)
    pltpu.sync_copy(r_hbm.at[pl.ds(base, RPW)], ridx_v)

    bufs0 = (a0, b0, c0, d0)
    bufs1 = (a1, b1, c1, d1)
    _issue(er3, ei3, rr3, ri3, eidx_v, ridx_v, 0, bufs0, sem0)

    def pair_body(i, carry):
        g0 = 2 * i
        g1 = g0 + 1
        _issue(er3, ei3, rr3, ri3, eidx_v, ridx_v, g1, bufs1, sem1)
        _drain(er3, bufs0, sem0)
        _compute(bufs0, or_v, oi_v, g0)

        @pl.when(i < N_GROUPS // 2 - 1)
        def _():
            _issue(er3, ei3, rr3, ri3, eidx_v, ridx_v, g0 + 2, bufs0, sem0)

        _drain(er3, bufs1, sem1)
        _compute(bufs1, or_v, oi_v, g1)

        @pl.when(g1 % GROUPS_PER_OUT == GROUPS_PER_OUT - 1)
        def _():
            off = base + (g1 // GROUPS_PER_OUT) * OUT_ROWS
            pltpu.sync_copy(or_v, out_r.at[pl.ds(off, OUT_ROWS)])
            pltpu.sync_copy(oi_v, out_i.at[pl.ds(off, OUT_ROWS)])

        return carry

    lax.fori_loop(0, N_GROUPS // 2, pair_body, 0)


@jax.jit
def kernel(e1, r, ent_real, ent_img, rel_real, rel_img):
    er3 = ent_real.reshape(NUM_ENTITIES // 8, 8, EMBED_DIM)
    ei3 = ent_img.reshape(NUM_ENTITIES // 8, 8, EMBED_DIM)
    rr3 = rel_real.reshape(NUM_RELATIONS // 8, 8, EMBED_DIM)
    ri3 = rel_img.reshape(NUM_RELATIONS // 8, 8, EMBED_DIM)
    mesh = plsc.VectorSubcoreMesh(core_axis_name="c", subcore_axis_name="s")
    out_shape = jax.ShapeDtypeStruct((BATCH, EMBED_DIM), jnp.float32)
    rowbuf = pltpu.VMEM((G, EMBED_DIM), jnp.float32)
    fn = pl.kernel(
        _body,
        out_type=(out_shape, out_shape),
        mesh=mesh,
        scratch_types=[
            pltpu.VMEM((RPW,), jnp.int32),
            pltpu.VMEM((RPW,), jnp.int32),
            rowbuf, rowbuf, rowbuf, rowbuf,
            rowbuf, rowbuf, rowbuf, rowbuf,
            pltpu.VMEM((OUT_ROWS, EMBED_DIM), jnp.float32),
            pltpu.VMEM((OUT_ROWS, EMBED_DIM), jnp.float32),
            pltpu.SemaphoreType.DMA,
            pltpu.SemaphoreType.DMA,
        ],
        compiler_params=pltpu.CompilerParams(
            use_tc_tiling_on_sc=True, needs_layout_passes=False),
    )
    return fn(e1, r, er3, ei3, rr3, ri3)
